# baseline (speedup 1.0000x reference)
import jax
import jax.numpy as jnp
from jax import lax
from jax.experimental import pallas as pl
from jax.experimental.pallas import tpu as pltpu

_D = 2116
_NNODE = 19
_NEDGE = 128
_HIGH = lax.Precision.HIGHEST


def _adj_body(ef_ref, a_ref):
    src = ef_ref[0, :]
    dst = ef_ref[1, :]
    cols = lax.broadcasted_iota(jnp.int32, (_NEDGE, _NNODE), 1)
    src_oh = (src[:, None] == cols).astype(jnp.float32)
    dst_oh = (dst[:, None] == cols).astype(jnp.float32)
    a_ref[...] = lax.dot_general(
        dst_oh, src_oh, (((0,), (0,)), ((), ())),
        preferred_element_type=jnp.float32, precision=_HIGH)


def _build_adjacency(edge_index):
    return pl.pallas_call(
        _adj_body,
        out_shape=jax.ShapeDtypeStruct((_NNODE, _NNODE), jnp.float32),
    )(edge_index)


def _mm_body(h_ref, w_ref, o_ref):
    o_ref[...] = jnp.dot(h_ref[...].astype(jnp.bfloat16),
                         w_ref[...].astype(jnp.bfloat16),
                         preferred_element_type=jnp.float32)


def _matmul(h, w, tn):
    m_rows = h.shape[0]
    grid = (pl.cdiv(_D, tn),)
    return pl.pallas_call(
        _mm_body,
        grid=grid,
        in_specs=[
            pl.BlockSpec((m_rows, _D), lambda j: (0, 0)),
            pl.BlockSpec((_D, tn), lambda j: (0, j)),
        ],
        out_specs=pl.BlockSpec((m_rows, tn), lambda j: (0, j)),
        out_shape=jax.ShapeDtypeStruct((m_rows, _D), jnp.float32),
    )(h, w)


def _mix_body(a_ref, m_ref, o_ref):
    o_ref[0] = jnp.dot(a_ref[...], m_ref[0],
                       preferred_element_type=jnp.float32, precision=_HIGH)


def _mix(a, m3):
    nb = m3.shape[0]
    return pl.pallas_call(
        _mix_body,
        grid=(nb,),
        in_specs=[
            pl.BlockSpec((_NNODE, _NNODE), lambda i: (0, 0)),
            pl.BlockSpec((1, _NNODE, _D), lambda i: (i, 0, 0)),
        ],
        out_specs=pl.BlockSpec((1, _NNODE, _D), lambda i: (i, 0, 0)),
        out_shape=jax.ShapeDtypeStruct((nb, _NNODE, _D), jnp.float32),
    )(a, m3)


def _gru_body(agg_ref, h_ref, hcol_ref, wih_ref, whh_ref, bih_ref, bhh_ref,
              o_ref):
    agg = agg_ref[...].astype(jnp.bfloat16)
    h = h_ref[...].astype(jnp.bfloat16)

    def gate(k):
        gi = jnp.dot(agg, wih_ref[k].astype(jnp.bfloat16),
                     preferred_element_type=jnp.float32)
        gh = jnp.dot(h, whh_ref[k].astype(jnp.bfloat16),
                     preferred_element_type=jnp.float32)
        return gi + bih_ref[k][None, :], gh + bhh_ref[k][None, :]

    i_r, h_r = gate(0)
    i_z, h_z = gate(1)
    i_n, h_n = gate(2)
    r = jax.nn.sigmoid(i_r + h_r)
    z = jax.nn.sigmoid(i_z + h_z)
    n = jnp.tanh(i_n + r * h_n)
    o_ref[...] = (1.0 - z) * n + z * hcol_ref[...]


def _gru(agg, h, wih3, whh3, bih2, bhh2, tn):
    m_rows = h.shape[0]
    grid = (pl.cdiv(_D, tn),)
    return pl.pallas_call(
        _gru_body,
        grid=grid,
        in_specs=[
            pl.BlockSpec((m_rows, _D), lambda j: (0, 0)),
            pl.BlockSpec((m_rows, _D), lambda j: (0, 0)),
            pl.BlockSpec((m_rows, tn), lambda j: (0, j)),
            pl.BlockSpec((3, _D, tn), lambda j: (0, 0, j)),
            pl.BlockSpec((3, _D, tn), lambda j: (0, 0, j)),
            pl.BlockSpec((3, tn), lambda j: (0, j)),
            pl.BlockSpec((3, tn), lambda j: (0, j)),
        ],
        out_specs=pl.BlockSpec((m_rows, tn), lambda j: (0, j)),
        out_shape=jax.ShapeDtypeStruct((m_rows, _D), jnp.float32),
    )(agg, h, h, wih3, whh3, bih2, bhh2)


def kernel(cnn_output, edge_index, weight, w_ih, w_hh, b_ih, b_hh,
           gnn_interations):
    del gnn_interations
    nbatch, nchan, hh, ww = cnn_output.shape
    x = cnn_output.reshape(nbatch * nchan, hh * ww)

    a = _build_adjacency(edge_index)

    out = x
    for l in range(weight.shape[0]):
        wih3 = w_ih[l].reshape(3, _D, _D).transpose(0, 2, 1)
        whh3 = w_hh[l].reshape(3, _D, _D).transpose(0, 2, 1)
        bih2 = b_ih[l].reshape(3, _D)
        bhh2 = b_hh[l].reshape(3, _D)
        h = x
        for i in range(weight.shape[1]):
            m = _matmul(h, weight[l, i], tn=256)
            agg3 = _mix(a, m.reshape(nbatch, nchan, _D))
            agg = agg3.reshape(nbatch * nchan, _D)
            h = _gru(agg, h, wih3, whh3, bih2, bhh2, tn=128)
        out = h
    return out.reshape(nbatch, nchan, hh, ww)


# speedup vs baseline: 1.6464x; 1.6464x over previous
import jax
import jax.numpy as jnp
from jax import lax
from jax.experimental import pallas as pl
from jax.experimental.pallas import tpu as pltpu

_D = 2116
_NNODE = 19
_NEDGE = 128
_NB = 64
_HIGH = lax.Precision.HIGHEST


def _adj_body(ef_ref, a_ref):
    src = ef_ref[0, :]
    dst = ef_ref[1, :]
    cols = lax.broadcasted_iota(jnp.int32, (_NEDGE, _NNODE), 1)
    src_oh = (src[:, None] == cols).astype(jnp.float32)
    dst_oh = (dst[:, None] == cols).astype(jnp.float32)
    a_ref[...] = lax.dot_general(
        dst_oh, src_oh, (((0,), (0,)), ((), ())),
        preferred_element_type=jnp.float32, precision=_HIGH)


def _build_adjacency(edge_index):
    return pl.pallas_call(
        _adj_body,
        out_shape=jax.ShapeDtypeStruct((_NNODE, _NNODE), jnp.float32),
    )(edge_index)


def _agg_body(h_ref, p_ref, w_ref, o_ref):
    t = jnp.dot(h_ref[...], w_ref[...].astype(jnp.bfloat16),
                preferred_element_type=jnp.float32)
    o_ref[...] = jnp.dot(p_ref[...], t.astype(jnp.bfloat16),
                         preferred_element_type=jnp.float32
                         ).astype(jnp.bfloat16)


def _agg(h_bf, p_bf, w, tn):
    m_rows = h_bf.shape[0]
    grid = (pl.cdiv(_D, tn),)
    return pl.pallas_call(
        _agg_body,
        grid=grid,
        in_specs=[
            pl.BlockSpec((m_rows, _D), lambda j: (0, 0)),
            pl.BlockSpec((m_rows, m_rows), lambda j: (0, 0)),
            pl.BlockSpec((_D, tn), lambda j: (0, j)),
        ],
        out_specs=pl.BlockSpec((m_rows, tn), lambda j: (0, j)),
        out_shape=jax.ShapeDtypeStruct((m_rows, _D), jnp.bfloat16),
    )(h_bf, p_bf, w)


def _gru_body(agg_ref, h_ref, hcol_ref, wih_ref, whh_ref, bih_ref, bhh_ref,
              o_ref, obf_ref):
    agg = agg_ref[...]
    h = h_ref[...]

    def gate(k):
        gi = jnp.dot(agg, wih_ref[k], preferred_element_type=jnp.float32)
        gh = jnp.dot(h, whh_ref[k], preferred_element_type=jnp.float32)
        return gi + bih_ref[k][None, :], gh + bhh_ref[k][None, :]

    i_r, h_r = gate(0)
    i_z, h_z = gate(1)
    i_n, h_n = gate(2)
    r = jax.nn.sigmoid(i_r + h_r)
    z = jax.nn.sigmoid(i_z + h_z)
    n = jnp.tanh(i_n + r * h_n)
    h_new = (1.0 - z) * n + z * hcol_ref[...]
    o_ref[...] = h_new
    obf_ref[...] = h_new.astype(jnp.bfloat16)


def _gru(agg_bf, h_bf, h_f32, wih3, whh3, bih2, bhh2, tn):
    m_rows = h_bf.shape[0]
    grid = (pl.cdiv(_D, tn),)
    return pl.pallas_call(
        _gru_body,
        grid=grid,
        in_specs=[
            pl.BlockSpec((m_rows, _D), lambda j: (0, 0)),
            pl.BlockSpec((m_rows, _D), lambda j: (0, 0)),
            pl.BlockSpec((m_rows, tn), lambda j: (0, j)),
            pl.BlockSpec((3, _D, tn), lambda j: (0, 0, j)),
            pl.BlockSpec((3, _D, tn), lambda j: (0, 0, j)),
            pl.BlockSpec((3, tn), lambda j: (0, j)),
            pl.BlockSpec((3, tn), lambda j: (0, j)),
        ],
        out_specs=[
            pl.BlockSpec((m_rows, tn), lambda j: (0, j)),
            pl.BlockSpec((m_rows, tn), lambda j: (0, j)),
        ],
        out_shape=[
            jax.ShapeDtypeStruct((m_rows, _D), jnp.float32),
            jax.ShapeDtypeStruct((m_rows, _D), jnp.bfloat16),
        ],
    )(agg_bf, h_bf, h_f32, wih3, whh3, bih2, bhh2)


def kernel(cnn_output, edge_index, weight, w_ih, w_hh, b_ih, b_hh,
           gnn_interations):
    del gnn_interations
    nbatch, nchan, hh, ww = cnn_output.shape
    x = cnn_output.reshape(nbatch * nchan, hh * ww)

    a = _build_adjacency(edge_index)
    p_bf = jnp.kron(jnp.eye(nbatch, dtype=jnp.float32), a).astype(jnp.bfloat16)

    out = x
    for l in range(weight.shape[0]):
        wih3 = (w_ih[l].reshape(3, _D, _D).transpose(0, 2, 1)
                .astype(jnp.bfloat16))
        whh3 = (w_hh[l].reshape(3, _D, _D).transpose(0, 2, 1)
                .astype(jnp.bfloat16))
        bih2 = b_ih[l].reshape(3, _D)
        bhh2 = b_hh[l].reshape(3, _D)
        h_f32 = x
        h_bf = x.astype(jnp.bfloat16)
        for i in range(weight.shape[1]):
            agg_bf = _agg(h_bf, p_bf, weight[l, i], tn=256)
            h_f32, h_bf = _gru(agg_bf, h_bf, h_f32, wih3, whh3, bih2, bhh2,
                               tn=256)
        out = h_f32
    return out.reshape(nbatch, nchan, hh, ww)


# speedup vs baseline: 2.6605x; 1.6159x over previous
import jax
import jax.numpy as jnp
from jax import lax
from jax.experimental import pallas as pl
from jax.experimental.pallas import tpu as pltpu

_D = 2116
_NNODE = 19
_NEDGE = 128
_NB = 64
_HIGH = lax.Precision.HIGHEST


def _adj_body(ef_ref, a_ref):
    src = ef_ref[0, :]
    dst = ef_ref[1, :]
    cols = lax.broadcasted_iota(jnp.int32, (_NEDGE, _NNODE), 1)
    src_oh = (src[:, None] == cols).astype(jnp.float32)
    dst_oh = (dst[:, None] == cols).astype(jnp.float32)
    a_ref[...] = lax.dot_general(
        dst_oh, src_oh, (((0,), (0,)), ((), ())),
        preferred_element_type=jnp.float32, precision=_HIGH)


def _build_adjacency(edge_index):
    return pl.pallas_call(
        _adj_body,
        out_shape=jax.ShapeDtypeStruct((_NNODE, _NNODE), jnp.float32),
    )(edge_index)


def _agg_body(h_ref, p_ref, w_ref, o_ref):
    t = jnp.dot(h_ref[...], w_ref[...].astype(jnp.bfloat16),
                preferred_element_type=jnp.float32)
    o_ref[...] = jnp.dot(p_ref[...], t.astype(jnp.bfloat16),
                         preferred_element_type=jnp.float32
                         ).astype(jnp.bfloat16)


def _agg(h_bf, p_bf, w, tn):
    m_rows = h_bf.shape[0]
    grid = (pl.cdiv(_D, tn),)
    return pl.pallas_call(
        _agg_body,
        grid=grid,
        in_specs=[
            pl.BlockSpec((m_rows, _D), lambda j: (0, 0)),
            pl.BlockSpec((m_rows, m_rows), lambda j: (0, 0)),
            pl.BlockSpec((_D, tn), lambda j: (0, j)),
        ],
        out_specs=pl.BlockSpec((m_rows, tn), lambda j: (0, j)),
        out_shape=jax.ShapeDtypeStruct((m_rows, _D), jnp.bfloat16),
    )(h_bf, p_bf, w)


def _gru_body(agg_ref, h_ref, hcol_ref, wih_ref, whh_ref, bih_ref, bhh_ref,
              o_ref, obf_ref):
    agg = agg_ref[...]
    h = h_ref[...]

    def gate(k):
        gi = lax.dot_general(agg, wih_ref[k], (((1,), (1,)), ((), ())),
                             preferred_element_type=jnp.float32)
        gh = lax.dot_general(h, whh_ref[k], (((1,), (1,)), ((), ())),
                             preferred_element_type=jnp.float32)
        return gi + bih_ref[k][None, :], gh + bhh_ref[k][None, :]

    i_r, h_r = gate(0)
    i_z, h_z = gate(1)
    i_n, h_n = gate(2)
    r = jax.nn.sigmoid(i_r + h_r)
    z = jax.nn.sigmoid(i_z + h_z)
    n = jnp.tanh(i_n + r * h_n)
    h_new = (1.0 - z) * n + z * hcol_ref[...]
    o_ref[...] = h_new
    obf_ref[...] = h_new.astype(jnp.bfloat16)


def _gru(agg_bf, h_bf, h_f32, wih3, whh3, bih2, bhh2, tn):
    m_rows = h_bf.shape[0]
    grid = (pl.cdiv(_D, tn),)
    return pl.pallas_call(
        _gru_body,
        grid=grid,
        in_specs=[
            pl.BlockSpec((m_rows, _D), lambda j: (0, 0)),
            pl.BlockSpec((m_rows, _D), lambda j: (0, 0)),
            pl.BlockSpec((m_rows, tn), lambda j: (0, j)),
            pl.BlockSpec((3, tn, _D), lambda j: (0, j, 0)),
            pl.BlockSpec((3, tn, _D), lambda j: (0, j, 0)),
            pl.BlockSpec((3, tn), lambda j: (0, j)),
            pl.BlockSpec((3, tn), lambda j: (0, j)),
        ],
        out_specs=[
            pl.BlockSpec((m_rows, tn), lambda j: (0, j)),
            pl.BlockSpec((m_rows, tn), lambda j: (0, j)),
        ],
        out_shape=[
            jax.ShapeDtypeStruct((m_rows, _D), jnp.float32),
            jax.ShapeDtypeStruct((m_rows, _D), jnp.bfloat16),
        ],
    )(agg_bf, h_bf, h_f32, wih3, whh3, bih2, bhh2)


def kernel(cnn_output, edge_index, weight, w_ih, w_hh, b_ih, b_hh,
           gnn_interations):
    del gnn_interations
    nbatch, nchan, hh, ww = cnn_output.shape
    x = cnn_output.reshape(nbatch * nchan, hh * ww)

    a = _build_adjacency(edge_index)
    p_bf = jnp.kron(jnp.eye(nbatch, dtype=jnp.float32), a).astype(jnp.bfloat16)

    out = x
    for l in range(weight.shape[0]):
        wih3 = w_ih[l].reshape(3, _D, _D).astype(jnp.bfloat16)
        whh3 = w_hh[l].reshape(3, _D, _D).astype(jnp.bfloat16)
        bih2 = b_ih[l].reshape(3, _D)
        bhh2 = b_hh[l].reshape(3, _D)
        h_f32 = x
        h_bf = x.astype(jnp.bfloat16)
        for i in range(weight.shape[1]):
            agg_bf = _agg(h_bf, p_bf, weight[l, i], tn=256)
            h_f32, h_bf = _gru(agg_bf, h_bf, h_f32, wih3, whh3, bih2, bhh2,
                               tn=256)
        out = h_f32
    return out.reshape(nbatch, nchan, hh, ww)
